# baseline (speedup 1.0000x reference)
import functools

import jax
import jax.numpy as jnp
from jax import lax
from jax.experimental import pallas as pl
from jax.experimental.pallas import tpu as pltpu
from jax.experimental.pallas import tpu_sc as plsc

_L = 16
_EPS = 1e-5
_IDX_A = 104
_IDX_B = 96


def kernel(actions, action_table, temporal_table, gamma, beta):
    B, T = actions.shape
    D = action_table.shape[1]
    nvec = D // _L

    info = plsc.get_sparse_core_info()
    nw = info.num_cores * info.num_subcores
    rows_per_w = B // nw

    t0s = list(range(0, (T // _L) * _L - _L + 1, _L)) + [T - _L]
    nb = len(t0s)
    inv_d = 1.0 / D

    mesh = plsc.VectorSubcoreMesh(core_axis_name="c", subcore_axis_name="s")

    @functools.partial(
        pl.kernel,
        mesh=mesh,
        compiler_params=pltpu.CompilerParams(
            needs_layout_passes=False, use_tc_tiling_on_sc=False),
        out_type=jax.ShapeDtypeStruct((B * T, D), jnp.float32),
        scratch_types=[
            pltpu.VMEM((_IDX_A,), jnp.int32),
            pltpu.VMEM((_IDX_B,), jnp.int32),
            pltpu.VMEM((T, D), jnp.float32),
            pltpu.VMEM((D, T), jnp.float32),
            pltpu.VMEM((D, nb * _L), jnp.float32),
            pltpu.VMEM((D,), jnp.float32),
            pltpu.VMEM((D,), jnp.float32),
            pltpu.SemaphoreType.DMA,
        ],
    )
    def k(actions_hbm, table_hbm, temporal_t_hbm, gamma_hbm, beta_hbm,
          out_hbm, idx_a, idx_b, rows_v, temp_t, y_t, g_v, b_v, sem):
        wid = lax.axis_index("s") * info.num_cores + lax.axis_index("c")
        pltpu.sync_copy(temporal_t_hbm, temp_t)
        pltpu.sync_copy(gamma_hbm, g_v)
        pltpu.sync_copy(beta_hbm, b_v)
        lane = lax.iota(jnp.int32, _L)
        tvecs = [t0 + lane for t0 in t0s]

        def chunk(r, carry):
            brow = wid * rows_per_w + r
            base = brow * T
            pltpu.sync_copy(actions_hbm.at[pl.ds(base, _IDX_A)], idx_a)
            pltpu.sync_copy(actions_hbm.at[pl.ds(base + _IDX_A, _IDX_B)],
                            idx_b)
            c1 = pltpu.async_copy(table_hbm.at[idx_a],
                                  rows_v.at[pl.ds(0, _IDX_A)], sem)
            c2 = pltpu.async_copy(table_hbm.at[idx_b],
                                  rows_v.at[pl.ds(_IDX_A, _IDX_B)], sem)
            c1.wait()
            c2.wait()

            def pass1(d, acc):
                sums, sqs = acc
                dvec = jnp.broadcast_to(d, (_L,))
                new_sums = []
                new_sqs = []
                for bi in range(nb):
                    x = plsc.load_gather(rows_v, [tvecs[bi], dvec])
                    y = x + temp_t[d, pl.ds(t0s[bi], _L)]
                    y_t[d, pl.ds(bi * _L, _L)] = y
                    new_sums.append(sums[bi] + y)
                    new_sqs.append(sqs[bi] + y * y)
                return tuple(new_sums), tuple(new_sqs)

            zeros = tuple(jnp.zeros((_L,), jnp.float32) for _ in range(nb))
            sums, sqs = lax.fori_loop(0, D, pass1, (zeros, zeros))

            means = []
            rstds = []
            for bi in range(nb):
                mean = sums[bi] * inv_d
                var = sqs[bi] * inv_d - mean * mean
                x = var + _EPS
                xi = lax.bitcast_convert_type(x, jnp.int32)
                yi = jnp.int32(0x5F3759DF) - (xi >> 1)
                rr = lax.bitcast_convert_type(yi, jnp.float32)
                rr = rr * (1.5 - 0.5 * x * rr * rr)
                rr = rr * (1.5 - 0.5 * x * rr * rr)
                means.append(mean)
                rstds.append(rr)

            def pass2(d, carry2):
                dvec = jnp.broadcast_to(d, (_L,))
                gs = plsc.load_gather(g_v, [dvec])
                bs = plsc.load_gather(b_v, [dvec])
                for bi in range(nb):
                    y = y_t[d, pl.ds(bi * _L, _L)]
                    o = (y - means[bi]) * rstds[bi] * gs + bs
                    plsc.store_scatter(rows_v, [tvecs[bi], dvec], o)
                return carry2

            lax.fori_loop(0, D, pass2, 0)
            pltpu.sync_copy(rows_v, out_hbm.at[pl.ds(base, T)])
            return carry

        lax.fori_loop(0, rows_per_w, chunk, 0)

    out = k(actions.reshape(-1), action_table, temporal_table.T,
            gamma, beta)
    return out.reshape(B, T, D)


# speedup vs baseline: 1.0732x; 1.0732x over previous
import functools

import jax
import jax.numpy as jnp
from jax import lax
from jax.experimental import pallas as pl
from jax.experimental.pallas import tpu as pltpu
from jax.experimental.pallas import tpu_sc as plsc

_L = 16
_EPS = 1e-5
_IDX_A = 104
_IDX_B = 96


def kernel(actions, action_table, temporal_table, gamma, beta):
    B, T = actions.shape
    D = action_table.shape[1]

    info = plsc.get_sparse_core_info()
    nw = info.num_cores * info.num_subcores
    rows_per_w = B // nw

    t0s = list(range(0, (T // _L) * _L - _L + 1, _L)) + [T - _L]
    nb = len(t0s)
    inv_d = 1.0 / D

    mesh = plsc.VectorSubcoreMesh(core_axis_name="c", subcore_axis_name="s")

    @functools.partial(
        pl.kernel,
        mesh=mesh,
        compiler_params=pltpu.CompilerParams(
            needs_layout_passes=False, use_tc_tiling_on_sc=False),
        out_type=jax.ShapeDtypeStruct((B * T, D), jnp.float32),
        scratch_types=[
            pltpu.VMEM((rows_per_w * T,), jnp.int32),
            pltpu.VMEM((T, D), jnp.float32),
            pltpu.VMEM((T, D), jnp.float32),
            pltpu.VMEM((D, T), jnp.float32),
            pltpu.VMEM((D, nb * _L), jnp.float32),
            pltpu.VMEM((D,), jnp.float32),
            pltpu.VMEM((D,), jnp.float32),
            pltpu.SemaphoreType.DMA,
            pltpu.SemaphoreType.DMA,
        ],
    )
    def k(actions_hbm, table_hbm, temporal_t_hbm, gamma_hbm, beta_hbm,
          out_hbm, idx_all, rows0, rows1, temp_t, y_t, g_v, b_v,
          sem_g, sem_o):
        wid = lax.axis_index("s") * info.num_cores + lax.axis_index("c")
        wbase = wid * rows_per_w * T
        pltpu.sync_copy(temporal_t_hbm, temp_t)
        pltpu.sync_copy(gamma_hbm, g_v)
        pltpu.sync_copy(beta_hbm, b_v)
        pltpu.sync_copy(actions_hbm.at[pl.ds(wbase, rows_per_w * T)],
                        idx_all)
        lane = lax.iota(jnp.int32, _L)
        tvecs = [t0 + lane for t0 in t0s]
        rows = (rows0, rows1)

        def issue_gather(r, buf):
            off = r * T
            c1 = pltpu.async_copy(
                table_hbm.at[idx_all.at[pl.ds(off, _IDX_A)]],
                buf.at[pl.ds(0, _IDX_A)], sem_g)
            c2 = pltpu.async_copy(
                table_hbm.at[idx_all.at[pl.ds(off + _IDX_A, _IDX_B)]],
                buf.at[pl.ds(_IDX_A, _IDX_B)], sem_g)
            return c1, c2

        def wait_gather():
            pltpu.make_async_copy(
                table_hbm.at[idx_all.at[pl.ds(0, _IDX_A)]],
                rows0.at[pl.ds(0, _IDX_A)], sem_g).wait()
            pltpu.make_async_copy(
                table_hbm.at[idx_all.at[pl.ds(0, _IDX_B)]],
                rows0.at[pl.ds(0, _IDX_B)], sem_g).wait()

        def compute(buf):
            def pass1(d, acc):
                sums, sqs = acc
                dvec = jnp.broadcast_to(d, (_L,))
                new_sums = []
                new_sqs = []
                for bi in range(nb):
                    x = plsc.load_gather(buf, [tvecs[bi], dvec])
                    y = x + temp_t[d, pl.ds(t0s[bi], _L)]
                    y_t[d, pl.ds(bi * _L, _L)] = y
                    new_sums.append(sums[bi] + y)
                    new_sqs.append(sqs[bi] + y * y)
                return tuple(new_sums), tuple(new_sqs)

            zeros = tuple(jnp.zeros((_L,), jnp.float32) for _ in range(nb))
            sums, sqs = lax.fori_loop(0, D, pass1, (zeros, zeros))

            means = []
            rstds = []
            for bi in range(nb):
                mean = sums[bi] * inv_d
                var = sqs[bi] * inv_d - mean * mean
                x = var + _EPS
                xi = lax.bitcast_convert_type(x, jnp.int32)
                yi = jnp.int32(0x5F3759DF) - (xi >> 1)
                rr = lax.bitcast_convert_type(yi, jnp.float32)
                rr = rr * (1.5 - 0.5 * x * rr * rr)
                rr = rr * (1.5 - 0.5 * x * rr * rr)
                means.append(mean)
                rstds.append(rr)

            def pass2(d, carry2):
                dvec = jnp.broadcast_to(d, (_L,))
                gs = plsc.load_gather(g_v, [dvec])
                bs = plsc.load_gather(b_v, [dvec])
                for bi in range(nb):
                    y = y_t[d, pl.ds(bi * _L, _L)]
                    o = (y - means[bi]) * rstds[bi] * gs + bs
                    plsc.store_scatter(buf, [tvecs[bi], dvec], o)
                return carry2

            lax.fori_loop(0, D, pass2, 0)

        def wait_out(buf):
            pltpu.make_async_copy(buf, out_hbm.at[pl.ds(0, T)], sem_o).wait()

        issue_gather(0, rows0)

        def outer(g2, _):
            for b in range(2):
                r = 2 * g2 + b
                wait_gather()
                if b == 0:
                    @pl.when(g2 >= 1)
                    def _():
                        wait_out(rows1)
                    issue_gather(r + 1, rows1)
                else:
                    wait_out(rows0)

                    @pl.when(g2 < (rows_per_w // 2) - 1)
                    def _():
                        issue_gather(r + 1, rows0)
                compute(rows[b])
                pltpu.async_copy(
                    rows[b], out_hbm.at[pl.ds(wbase + r * T, T)], sem_o)
            return 0

        lax.fori_loop(0, rows_per_w // 2, outer, 0)
        wait_out(rows1)

    out = k(actions.reshape(-1), action_table, temporal_table.T,
            gamma, beta)
    return out.reshape(B, T, D)
